# baseline (speedup 1.0000x reference)
import functools

import jax
import jax.numpy as jnp
from jax import lax
from jax.experimental import pallas as pl
from jax.experimental.pallas import tpu as pltpu
from jax.experimental.pallas import tpu_sc as plsc

N = 50000
HID = 64
FF = 128
OUT_DIM = 80

NC = 2
NS = 16
L = 16

BN = 2000
GRID = N // BN

CHUNK = 128
EPAD_UNIT = NS * CHUNK * 8
E_TOTAL = 800000
EPAD = ((E_TOTAL + EPAD_UNIT - 1) // EPAD_UNIT) * EPAD_UNIT
CROWS = EPAD // CHUNK
CROWS_TILE = CROWS // NS
STAGES = 49
CH_Q = CROWS_TILE // STAGES
GRP = 2
NGRP = CH_Q // GRP
GROWS = GRP * CHUNK
NSLOT = 3

ACC_TILE = 3136
NACC = NS * ACC_TILE
ZB_ROWS = 448
ZB_STEPS = ACC_TILE // ZB_ROWS
WB_ROWS = 224
WB_STEPS = ACC_TILE // WB_ROWS


def _dot(a, b):
    return jnp.dot(a, b, preferred_element_type=jnp.float32)


def _colmax8(v):
    return jnp.broadcast_to(jnp.max(v, axis=0, keepdims=True), (8, HID))


def _k0_body(x_ref, w_ref, b_ref, h_ref, m_ref):
    j = pl.program_id(0)
    h = _dot(x_ref[...], w_ref[...]) + b_ref[0:1, :]
    h_ref[...] = h
    bm = _colmax8(jax.nn.relu(h) + 1e-7)

    @pl.when(j == 0)
    def _():
        m_ref[...] = bm

    @pl.when(j > 0)
    def _():
        m_ref[...] = jnp.maximum(m_ref[...], bm)


def _run_k0(xp, Wlp, bl2):
    return pl.pallas_call(
        _k0_body,
        grid=(GRID,),
        in_specs=[
            pl.BlockSpec((BN, 8), lambda j: (j, 0)),
            pl.BlockSpec((8, HID), lambda j: (0, 0)),
            pl.BlockSpec((8, HID), lambda j: (0, 0)),
        ],
        out_specs=[
            pl.BlockSpec((BN, HID), lambda j: (j, 0)),
            pl.BlockSpec((8, HID), lambda j: (0, 0)),
        ],
        out_shape=[
            jax.ShapeDtypeStruct((N, HID), jnp.float32),
            jax.ShapeDtypeStruct((8, HID), jnp.float32),
        ],
    )(xp, Wlp, bl2)


def _tbl_body(h_ref, m_ref, t_ref):
    v = jax.nn.relu(h_ref[...]) + 1e-7
    w = jnp.exp(v - m_ref[0:1, :])
    wv = w * v
    parts = []
    for cb in range(4):
        parts.append(w[:, cb * L:(cb + 1) * L])
        parts.append(wv[:, cb * L:(cb + 1) * L])
    t_ref[...] = jnp.concatenate(parts, axis=1)


def _run_tbl(h, M):
    return pl.pallas_call(
        _tbl_body,
        grid=(GRID,),
        in_specs=[
            pl.BlockSpec((BN, HID), lambda j: (j, 0)),
            pl.BlockSpec((8, HID), lambda j: (0, 0)),
        ],
        out_specs=[pl.BlockSpec((BN, 2 * HID), lambda j: (j, 0))],
        out_shape=[jax.ShapeDtypeStruct((N, 2 * HID), jnp.float32)],
    )(h, M)[0]


def _sc_body(src_ref, dst_ref, tab_ref, agg_ref,
             accC, sidx, didx, gb, semg, sems):
    c = lax.axis_index("c")
    s = lax.axis_index("s")
    row0 = s * ACC_TILE
    chunk0 = s * CROWS_TILE

    def _fire_gathers(g, slot):
        pltpu.async_copy(tab_ref.at[sidx.at[pl.ds(g * GROWS, GROWS)]],
                         gb.at[pl.ds(slot * GROWS, GROWS)], semg)

    def _fire_scatters(g, slot):
        off = slot * GROWS
        for j in range(GRP):
            r = g * GRP + j
            pltpu.async_copy(gb.at[pl.ds(off + j * CHUNK, CHUNK)],
                             accC.at[didx.at[r]], sems, add=True)

    def _drain(sem, rows):
        pltpu.make_async_copy(tab_ref.at[pl.ds(0, rows)],
                              gb.at[pl.ds(0, rows)], sem).wait()

    def _edges(cb):
        def stage(q, _):
            r0 = chunk0 + q * CH_Q
            pltpu.sync_copy(src_ref.at[pl.ds(r0 * CHUNK, CH_Q * CHUNK)], sidx)
            pltpu.sync_copy(dst_ref.at[pl.ds(r0, CH_Q)], didx)
            for i in range(CH_Q * CHUNK // L):
                sl = sidx[pl.ds(i * L, L)]
                sidx[pl.ds(i * L, L)] = sl * 4 + cb
            _fire_gathers(0, 0)
            _fire_gathers(1, 1)

            def grp(g, _):
                slot = lax.rem(g, NSLOT)
                _drain(semg, GROWS)
                _fire_scatters(g, slot)

                @pl.when(g > 0)
                def _():
                    _drain(sems, GROWS)

                @pl.when(g < NGRP - 2)
                def _():
                    _fire_gathers(g + 2, lax.rem(g + 2, NSLOT))
                return _
            lax.fori_loop(0, NGRP, grp, None)
            _drain(sems, GROWS)
            return _
        lax.fori_loop(0, STAGES, stage, None)

    def _zero():
        def zf(i, _):
            gb[i, 0:L] = jnp.zeros((L,), jnp.float32)
            gb[i, L:2 * L] = jnp.zeros((L,), jnp.float32)
            return _
        lax.fori_loop(0, ZB_ROWS, zf, None)
        for k in range(ZB_STEPS):
            pltpu.async_copy(gb.at[pl.ds(0, ZB_ROWS)],
                             accC.at[pl.ds(row0 + k * ZB_ROWS, ZB_ROWS)], semg)
        _drain(semg, ACC_TILE)

    def _writeback(cb):
        pltpu.async_copy(accC.at[pl.ds(row0, WB_ROWS)],
                         gb.at[pl.ds(0, WB_ROWS)], semg)

        def wr(k, _):
            off = lax.rem(k, 2) * GROWS

            @pl.when(k > 0)
            def _():
                _drain(sems, WB_ROWS)

            @pl.when(k < WB_STEPS - 1)
            def _():
                pltpu.async_copy(
                    accC.at[pl.ds(row0 + (k + 1) * WB_ROWS, WB_ROWS)],
                    gb.at[pl.ds((lax.rem(k, 2) ^ 1) * GROWS, WB_ROWS)], semg)
            _drain(semg, WB_ROWS)
            pltpu.async_copy(
                gb.at[pl.ds(off, WB_ROWS)],
                agg_ref.at[pl.ds(row0 + k * WB_ROWS, WB_ROWS),
                           pl.ds(cb * 2 * L, 2 * L)], sems)
            return _
        lax.fori_loop(0, WB_STEPS, wr, None)
        _drain(sems, WB_ROWS)

    for phase in range(2):
        _zero()
        plsc.subcore_barrier()

        for cc in range(NC):
            cb = 2 * cc + phase

            @pl.when(c == cc)
            def _(cb=cb):
                _edges(cb)
        plsc.subcore_barrier()

        for cc in range(NC):
            cb = 2 * cc + phase

            @pl.when(c == cc)
            def _(cb=cb):
                _writeback(cb)
        plsc.subcore_barrier()


def _run_sc(src2d, dst2d, tab4):
    f = pl.kernel(
        _sc_body,
        out_type=[jax.ShapeDtypeStruct((NACC, 8 * L), jnp.float32)],
        mesh=plsc.VectorSubcoreMesh(core_axis_name="c", subcore_axis_name="s"),
        compiler_params=pltpu.CompilerParams(use_tc_tiling_on_sc=False),
        scratch_types=[
            pltpu.VMEM_SHARED((NACC, 2 * L), jnp.float32),
            pltpu.VMEM((CH_Q * CHUNK,), jnp.int32),
            pltpu.VMEM((CH_Q, CHUNK), jnp.int32),
            pltpu.VMEM((NSLOT * GROWS, 2 * L), jnp.float32),
            pltpu.SemaphoreType.DMA,
            pltpu.SemaphoreType.DMA,
        ],
    )
    return f(src2d, dst2d, tab4)[0]


def _aggr_from(agg_ref, m_ref):
    eps = jnp.maximum(1e-16 * jnp.exp(-m_ref[0:1, :]), 1e-38)
    a = agg_ref[...]
    return jnp.concatenate(
        [a[:, cb * 2 * L + L:cb * 2 * L + 2 * L]
         / (a[:, cb * 2 * L:cb * 2 * L + L] + eps[:, cb * L:(cb + 1) * L])
         for cb in range(4)], axis=1)


def _layer_body(h_ref, m_ref, w1_ref, b1_ref, w2_ref, b2_ref, agg_ref,
                h_out, m_out, *, first):
    j = pl.program_id(0)
    aggr = _aggr_from(agg_ref, m_ref)
    h = h_ref[...]
    base = h if first else jax.nn.relu(h)
    u = base + aggr
    t = _dot(jax.nn.relu(_dot(u, w1_ref[...]) + b1_ref[0:1, :]),
             w2_ref[...]) + b2_ref[0:1, :]
    hnew = jax.nn.relu(t) if first else h + t
    h_out[...] = hnew
    bm = _colmax8(jax.nn.relu(hnew) + 1e-7)

    @pl.when(j == 0)
    def _():
        m_out[...] = bm

    @pl.when(j > 0)
    def _():
        m_out[...] = jnp.maximum(m_out[...], bm)


def _run_layer(h, M, W1, b1, W2, b2, agg, first):
    return pl.pallas_call(
        functools.partial(_layer_body, first=first),
        grid=(GRID,),
        in_specs=[
            pl.BlockSpec((BN, HID), lambda j: (j, 0)),
            pl.BlockSpec((8, HID), lambda j: (0, 0)),
            pl.BlockSpec((HID, FF), lambda j: (0, 0)),
            pl.BlockSpec((8, FF), lambda j: (0, 0)),
            pl.BlockSpec((FF, HID), lambda j: (0, 0)),
            pl.BlockSpec((8, HID), lambda j: (0, 0)),
            pl.BlockSpec((BN, 8 * L), lambda j: (j, 0)),
        ],
        out_specs=[
            pl.BlockSpec((BN, HID), lambda j: (j, 0)),
            pl.BlockSpec((8, HID), lambda j: (0, 0)),
        ],
        out_shape=[
            jax.ShapeDtypeStruct((N, HID), jnp.float32),
            jax.ShapeDtypeStruct((8, HID), jnp.float32),
        ],
    )(h, M, W1, b1, W2, b2, agg)


def _final_body(h_ref, m_ref, w1_ref, b1_ref, w2_ref, b2_ref,
                wh1_ref, bh1_ref, wh2_ref, bh2_ref, agg_ref,
                out_ref, pool_ref):
    j = pl.program_id(0)
    aggr = _aggr_from(agg_ref, m_ref)
    h = h_ref[...]
    u = jax.nn.relu(h) + aggr
    t = _dot(jax.nn.relu(_dot(u, w1_ref[...]) + b1_ref[0:1, :]),
             w2_ref[...]) + b2_ref[0:1, :]
    hnew = h + t
    bm = jnp.broadcast_to(jnp.max(hnew, axis=0, keepdims=True), (8, HID))

    @pl.when(j == 0)
    def _():
        pool_ref[...] = bm

    @pl.when(j > 0)
    def _():
        pool_ref[...] = jnp.maximum(pool_ref[...], bm)

    @pl.when(j == GRID - 1)
    def _():
        pooled = pool_ref[...]
        z = jax.nn.relu(_dot(pooled, wh1_ref[...]) + bh1_ref[0:1, :])
        out_ref[...] = _dot(z, wh2_ref[...]) + bh2_ref[0:1, :]


def _run_final(h, M, W1, b1, W2, b2, Wh1, bh1, Wh2, bh2, agg):
    return pl.pallas_call(
        _final_body,
        grid=(GRID,),
        in_specs=[
            pl.BlockSpec((BN, HID), lambda j: (j, 0)),
            pl.BlockSpec((8, HID), lambda j: (0, 0)),
            pl.BlockSpec((HID, FF), lambda j: (0, 0)),
            pl.BlockSpec((8, FF), lambda j: (0, 0)),
            pl.BlockSpec((FF, HID), lambda j: (0, 0)),
            pl.BlockSpec((8, HID), lambda j: (0, 0)),
            pl.BlockSpec((HID, HID), lambda j: (0, 0)),
            pl.BlockSpec((8, HID), lambda j: (0, 0)),
            pl.BlockSpec((HID, OUT_DIM), lambda j: (0, 0)),
            pl.BlockSpec((8, OUT_DIM), lambda j: (0, 0)),
            pl.BlockSpec((BN, 8 * L), lambda j: (j, 0)),
        ],
        out_specs=[pl.BlockSpec((8, OUT_DIM), lambda j: (0, 0))],
        out_shape=[jax.ShapeDtypeStruct((8, OUT_DIM), jnp.float32)],
        scratch_shapes=[pltpu.VMEM((8, HID), jnp.float32)],
    )(h, M, W1, b1, W2, b2, Wh1, bh1, Wh2, bh2, agg)


def _b8(b):
    return jnp.broadcast_to(b[None, :], (8, b.shape[0]))


def kernel(x, edge_index, Wl, bl, Win1, bin1, Win2, bin2,
           W0_1, b0_1, W0_2, b0_2, W1_1, b1_1, W1_2, b1_2,
           W2_1, b2_1, W2_2, b2_2, Wh1, bh1, Wh2, bh2):
    xp = jnp.pad(x, ((0, 0), (0, 2)))
    Wlp = jnp.pad(Wl, ((0, 2), (0, 0)))
    src = edge_index[0]
    dst = edge_index[1]
    pad = EPAD - src.shape[0]
    src1d = jnp.concatenate([src, jnp.zeros((pad,), jnp.int32)])
    dst2d = jnp.concatenate(
        [dst, jnp.full((pad,), N, jnp.int32)]).reshape(CROWS, CHUNK)

    h, M = _run_k0(xp, Wlp, _b8(bl))

    layers = [
        (Win1, bin1, Win2, bin2),
        (W0_1, b0_1, W0_2, b0_2),
        (W1_1, b1_1, W1_2, b1_2),
        (W2_1, b2_1, W2_2, b2_2),
    ]
    for li, (W1, b1, W2, b2) in enumerate(layers):
        tab = _run_tbl(h, M)
        tab4 = tab.reshape(4 * N, 2 * L)
        agg = _run_sc(src1d, dst2d, tab4)
        if li < 3:
            h, M = _run_layer(h, M, W1, _b8(b1), W2, _b8(b2), agg,
                              first=(li == 0))
        else:
            out8 = _run_final(h, M, W1, _b8(b1), W2, _b8(b2),
                              Wh1, _b8(bh1), Wh2, _b8(bh2), agg)[0]
    return out8[0:1, :]


# speedup vs baseline: 1.1052x; 1.1052x over previous
import functools

import jax
import jax.numpy as jnp
from jax import lax
from jax.experimental import pallas as pl
from jax.experimental.pallas import tpu as pltpu
from jax.experimental.pallas import tpu_sc as plsc

N = 50000
HID = 64
FF = 128
OUT_DIM = 80

NC = 2
NS = 16
L = 16

BN = 2000
GRID = N // BN

CHUNK = 128
EPAD_UNIT = NS * CHUNK * 8
E_TOTAL = 800000
EPAD = ((E_TOTAL + EPAD_UNIT - 1) // EPAD_UNIT) * EPAD_UNIT
CROWS = EPAD // CHUNK
CROWS_TILE = CROWS // NS
STAGES = 28
CH_Q = CROWS_TILE // STAGES
GRP = 2
NGRP = CH_Q // GRP
GROWS = GRP * CHUNK
NSLOT = 2

ACC_TILE = 3136
NACC = NS * ACC_TILE
ZB_ROWS = 448
ZB_STEPS = ACC_TILE // ZB_ROWS
WB_ROWS = 224
WB_STEPS = ACC_TILE // WB_ROWS


def _dot(a, b):
    return jnp.dot(a, b, preferred_element_type=jnp.float32)


def _colmax8(v):
    return jnp.broadcast_to(jnp.max(v, axis=0, keepdims=True), (8, HID))


def _k0_body(x_ref, w_ref, b_ref, h_ref, m_ref):
    j = pl.program_id(0)
    h = _dot(x_ref[...], w_ref[...]) + b_ref[0:1, :]
    h_ref[...] = h
    bm = _colmax8(jax.nn.relu(h) + 1e-7)

    @pl.when(j == 0)
    def _():
        m_ref[...] = bm

    @pl.when(j > 0)
    def _():
        m_ref[...] = jnp.maximum(m_ref[...], bm)


def _run_k0(xp, Wlp, bl2):
    return pl.pallas_call(
        _k0_body,
        grid=(GRID,),
        in_specs=[
            pl.BlockSpec((BN, 8), lambda j: (j, 0)),
            pl.BlockSpec((8, HID), lambda j: (0, 0)),
            pl.BlockSpec((8, HID), lambda j: (0, 0)),
        ],
        out_specs=[
            pl.BlockSpec((BN, HID), lambda j: (j, 0)),
            pl.BlockSpec((8, HID), lambda j: (0, 0)),
        ],
        out_shape=[
            jax.ShapeDtypeStruct((N, HID), jnp.float32),
            jax.ShapeDtypeStruct((8, HID), jnp.float32),
        ],
    )(xp, Wlp, bl2)


def _tbl_body(h_ref, m_ref, t_ref):
    v = jax.nn.relu(h_ref[...]) + 1e-7
    w = jnp.exp(v - m_ref[0:1, :])
    wv = w * v
    parts = []
    for cb in range(4):
        parts.append(w[:, cb * L:(cb + 1) * L])
        parts.append(wv[:, cb * L:(cb + 1) * L])
    t_ref[...] = jnp.concatenate(parts, axis=1)


def _run_tbl(h, M):
    return pl.pallas_call(
        _tbl_body,
        grid=(GRID,),
        in_specs=[
            pl.BlockSpec((BN, HID), lambda j: (j, 0)),
            pl.BlockSpec((8, HID), lambda j: (0, 0)),
        ],
        out_specs=[pl.BlockSpec((BN, 2 * HID), lambda j: (j, 0))],
        out_shape=[jax.ShapeDtypeStruct((N, 2 * HID), jnp.float32)],
    )(h, M)[0]


def _sc_body(src_ref, dst_ref, tab_ref, agg_ref,
             accC, sidx, didx, gb, semg, sems, semi):
    c = lax.axis_index("c")
    s = lax.axis_index("s")
    row0 = s * ACC_TILE
    chunk0 = s * CROWS_TILE

    def _fire_gathers(qs, g, slot):
        off = slot * GROWS
        for j in range(GRP):
            r = qs * CH_Q + g * GRP + j
            pltpu.async_copy(tab_ref.at[sidx.at[r]],
                             gb.at[pl.ds(off + j * CHUNK, CHUNK)], semg)

    def _fire_scatters(qs, g, slot):
        off = slot * GROWS
        for j in range(GRP):
            r = qs * CH_Q + g * GRP + j
            pltpu.async_copy(gb.at[pl.ds(off + j * CHUNK, CHUNK)],
                             accC.at[didx.at[r]], sems, add=True)

    def _drain(sem, rows):
        pltpu.make_async_copy(tab_ref.at[pl.ds(0, rows)],
                              gb.at[pl.ds(0, rows)], sem).wait()

    def _fire_idx(q, qs):
        r0 = chunk0 + q * CH_Q
        pltpu.async_copy(src_ref.at[pl.ds(r0, CH_Q)],
                         sidx.at[pl.ds(qs * CH_Q, CH_Q)], semi)
        pltpu.async_copy(dst_ref.at[pl.ds(r0, CH_Q)],
                         didx.at[pl.ds(qs * CH_Q, CH_Q)], semi)

    def _drain_idx():
        pltpu.make_async_copy(src_ref.at[pl.ds(0, CH_Q)],
                              sidx.at[pl.ds(0, CH_Q)], semi).wait()
        pltpu.make_async_copy(dst_ref.at[pl.ds(0, CH_Q)],
                              didx.at[pl.ds(0, CH_Q)], semi).wait()

    def _xform(qs, cb):
        for i in range(CH_Q):
            for j2 in range(CHUNK // L):
                sl = sidx[qs * CH_Q + i, pl.ds(j2 * L, L)]
                sidx[qs * CH_Q + i, pl.ds(j2 * L, L)] = sl * 4 + cb

    def _edges(cb):
        _fire_idx(0, 0)

        def stage(q, _):
            qs = lax.rem(q, 2)

            @pl.when(q < STAGES - 1)
            def _():
                _fire_idx(q + 1, 1 - qs)
            _drain_idx()

            @pl.when(qs == 0)
            def _():
                _xform(0, cb)

            @pl.when(qs == 1)
            def _():
                _xform(1, cb)
            _fire_gathers(qs, 0, 0)

            def grp(g, _):
                slot = lax.rem(g, NSLOT)

                @pl.when(g > 0)
                def _():
                    _drain(sems, GROWS)

                @pl.when(g < NGRP - 1)
                def _():
                    _fire_gathers(qs, g + 1, 1 - slot)
                _drain(semg, GROWS)
                _fire_scatters(qs, g, slot)
                return _
            lax.fori_loop(0, NGRP, grp, None)
            _drain(sems, GROWS)
            return _
        lax.fori_loop(0, STAGES, stage, None)

    def _zero():
        def zf(i, _):
            gb[i, 0:L] = jnp.zeros((L,), jnp.float32)
            gb[i, L:2 * L] = jnp.zeros((L,), jnp.float32)
            return _
        lax.fori_loop(0, ZB_ROWS, zf, None)
        for k in range(ZB_STEPS):
            pltpu.async_copy(gb.at[pl.ds(0, ZB_ROWS)],
                             accC.at[pl.ds(row0 + k * ZB_ROWS, ZB_ROWS)], semg)
        _drain(semg, ACC_TILE)

    def _writeback(cb):
        pltpu.async_copy(accC.at[pl.ds(row0, WB_ROWS)],
                         gb.at[pl.ds(0, WB_ROWS)], semg)

        def wr(k, _):
            off = lax.rem(k, 2) * GROWS

            @pl.when(k > 0)
            def _():
                _drain(sems, WB_ROWS)

            @pl.when(k < WB_STEPS - 1)
            def _():
                pltpu.async_copy(
                    accC.at[pl.ds(row0 + (k + 1) * WB_ROWS, WB_ROWS)],
                    gb.at[pl.ds((lax.rem(k, 2) ^ 1) * GROWS, WB_ROWS)], semg)
            _drain(semg, WB_ROWS)
            pltpu.async_copy(
                gb.at[pl.ds(off, WB_ROWS)],
                agg_ref.at[pl.ds(row0 + k * WB_ROWS, WB_ROWS),
                           pl.ds(cb * 2 * L, 2 * L)], sems)
            return _
        lax.fori_loop(0, WB_STEPS, wr, None)
        _drain(sems, WB_ROWS)

    for phase in range(2):
        _zero()
        plsc.subcore_barrier()

        for cc in range(NC):
            cb = 2 * cc + phase

            @pl.when(c == cc)
            def _(cb=cb):
                _edges(cb)
        plsc.subcore_barrier()

        for cc in range(NC):
            cb = 2 * cc + phase

            @pl.when(c == cc)
            def _(cb=cb):
                _writeback(cb)
        plsc.subcore_barrier()


def _run_sc(src2d, dst2d, tab4):
    f = pl.kernel(
        _sc_body,
        out_type=[jax.ShapeDtypeStruct((NACC, 8 * L), jnp.float32)],
        mesh=plsc.VectorSubcoreMesh(core_axis_name="c", subcore_axis_name="s"),
        compiler_params=pltpu.CompilerParams(use_tc_tiling_on_sc=False),
        scratch_types=[
            pltpu.VMEM_SHARED((NACC, 2 * L), jnp.float32),
            pltpu.VMEM((2 * CH_Q, CHUNK), jnp.int32),
            pltpu.VMEM((2 * CH_Q, CHUNK), jnp.int32),
            pltpu.VMEM((NSLOT * GROWS, 2 * L), jnp.float32),
            pltpu.SemaphoreType.DMA,
            pltpu.SemaphoreType.DMA,
            pltpu.SemaphoreType.DMA,
        ],
    )
    return f(src2d, dst2d, tab4)[0]


def _aggr_from(agg_ref, m_ref):
    eps = jnp.maximum(1e-16 * jnp.exp(-m_ref[0:1, :]), 1e-38)
    a = agg_ref[...]
    return jnp.concatenate(
        [a[:, cb * 2 * L + L:cb * 2 * L + 2 * L]
         / (a[:, cb * 2 * L:cb * 2 * L + L] + eps[:, cb * L:(cb + 1) * L])
         for cb in range(4)], axis=1)


def _layer_body(h_ref, m_ref, w1_ref, b1_ref, w2_ref, b2_ref, agg_ref,
                h_out, m_out, *, first):
    j = pl.program_id(0)
    aggr = _aggr_from(agg_ref, m_ref)
    h = h_ref[...]
    base = h if first else jax.nn.relu(h)
    u = base + aggr
    t = _dot(jax.nn.relu(_dot(u, w1_ref[...]) + b1_ref[0:1, :]),
             w2_ref[...]) + b2_ref[0:1, :]
    hnew = jax.nn.relu(t) if first else h + t
    h_out[...] = hnew
    bm = _colmax8(jax.nn.relu(hnew) + 1e-7)

    @pl.when(j == 0)
    def _():
        m_out[...] = bm

    @pl.when(j > 0)
    def _():
        m_out[...] = jnp.maximum(m_out[...], bm)


def _run_layer(h, M, W1, b1, W2, b2, agg, first):
    return pl.pallas_call(
        functools.partial(_layer_body, first=first),
        grid=(GRID,),
        in_specs=[
            pl.BlockSpec((BN, HID), lambda j: (j, 0)),
            pl.BlockSpec((8, HID), lambda j: (0, 0)),
            pl.BlockSpec((HID, FF), lambda j: (0, 0)),
            pl.BlockSpec((8, FF), lambda j: (0, 0)),
            pl.BlockSpec((FF, HID), lambda j: (0, 0)),
            pl.BlockSpec((8, HID), lambda j: (0, 0)),
            pl.BlockSpec((BN, 8 * L), lambda j: (j, 0)),
        ],
        out_specs=[
            pl.BlockSpec((BN, HID), lambda j: (j, 0)),
            pl.BlockSpec((8, HID), lambda j: (0, 0)),
        ],
        out_shape=[
            jax.ShapeDtypeStruct((N, HID), jnp.float32),
            jax.ShapeDtypeStruct((8, HID), jnp.float32),
        ],
    )(h, M, W1, b1, W2, b2, agg)


def _final_body(h_ref, m_ref, w1_ref, b1_ref, w2_ref, b2_ref,
                wh1_ref, bh1_ref, wh2_ref, bh2_ref, agg_ref,
                out_ref, pool_ref):
    j = pl.program_id(0)
    aggr = _aggr_from(agg_ref, m_ref)
    h = h_ref[...]
    u = jax.nn.relu(h) + aggr
    t = _dot(jax.nn.relu(_dot(u, w1_ref[...]) + b1_ref[0:1, :]),
             w2_ref[...]) + b2_ref[0:1, :]
    hnew = h + t
    bm = jnp.broadcast_to(jnp.max(hnew, axis=0, keepdims=True), (8, HID))

    @pl.when(j == 0)
    def _():
        pool_ref[...] = bm

    @pl.when(j > 0)
    def _():
        pool_ref[...] = jnp.maximum(pool_ref[...], bm)

    @pl.when(j == GRID - 1)
    def _():
        pooled = pool_ref[...]
        z = jax.nn.relu(_dot(pooled, wh1_ref[...]) + bh1_ref[0:1, :])
        out_ref[...] = _dot(z, wh2_ref[...]) + bh2_ref[0:1, :]


def _run_final(h, M, W1, b1, W2, b2, Wh1, bh1, Wh2, bh2, agg):
    return pl.pallas_call(
        _final_body,
        grid=(GRID,),
        in_specs=[
            pl.BlockSpec((BN, HID), lambda j: (j, 0)),
            pl.BlockSpec((8, HID), lambda j: (0, 0)),
            pl.BlockSpec((HID, FF), lambda j: (0, 0)),
            pl.BlockSpec((8, FF), lambda j: (0, 0)),
            pl.BlockSpec((FF, HID), lambda j: (0, 0)),
            pl.BlockSpec((8, HID), lambda j: (0, 0)),
            pl.BlockSpec((HID, HID), lambda j: (0, 0)),
            pl.BlockSpec((8, HID), lambda j: (0, 0)),
            pl.BlockSpec((HID, OUT_DIM), lambda j: (0, 0)),
            pl.BlockSpec((8, OUT_DIM), lambda j: (0, 0)),
            pl.BlockSpec((BN, 8 * L), lambda j: (j, 0)),
        ],
        out_specs=[pl.BlockSpec((8, OUT_DIM), lambda j: (0, 0))],
        out_shape=[jax.ShapeDtypeStruct((8, OUT_DIM), jnp.float32)],
        scratch_shapes=[pltpu.VMEM((8, HID), jnp.float32)],
    )(h, M, W1, b1, W2, b2, Wh1, bh1, Wh2, bh2, agg)


def _b8(b):
    return jnp.broadcast_to(b[None, :], (8, b.shape[0]))


def kernel(x, edge_index, Wl, bl, Win1, bin1, Win2, bin2,
           W0_1, b0_1, W0_2, b0_2, W1_1, b1_1, W1_2, b1_2,
           W2_1, b2_1, W2_2, b2_2, Wh1, bh1, Wh2, bh2):
    xp = jnp.pad(x, ((0, 0), (0, 2)))
    Wlp = jnp.pad(Wl, ((0, 2), (0, 0)))
    src = edge_index[0]
    dst = edge_index[1]
    pad = EPAD - src.shape[0]
    src2d = jnp.concatenate(
        [src, jnp.zeros((pad,), jnp.int32)]).reshape(CROWS, CHUNK)
    dst2d = jnp.concatenate(
        [dst, jnp.full((pad,), N, jnp.int32)]).reshape(CROWS, CHUNK)

    h, M = _run_k0(xp, Wlp, _b8(bl))

    layers = [
        (Win1, bin1, Win2, bin2),
        (W0_1, b0_1, W0_2, b0_2),
        (W1_1, b1_1, W1_2, b1_2),
        (W2_1, b2_1, W2_2, b2_2),
    ]
    for li, (W1, b1, W2, b2) in enumerate(layers):
        tab = _run_tbl(h, M)
        tab4 = tab.reshape(4 * N, 2 * L)
        agg = _run_sc(src2d, dst2d, tab4)
        if li < 3:
            h, M = _run_layer(h, M, W1, _b8(b1), W2, _b8(b2), agg,
                              first=(li == 0))
        else:
            out8 = _run_final(h, M, W1, _b8(b1), W2, _b8(b2),
                              Wh1, _b8(bh1), Wh2, _b8(bh2), agg)[0]
    return out8[0:1, :]


# speedup vs baseline: 1.1224x; 1.0156x over previous
import functools

import jax
import jax.numpy as jnp
from jax import lax
from jax.experimental import pallas as pl
from jax.experimental.pallas import tpu as pltpu
from jax.experimental.pallas import tpu_sc as plsc

N = 50000
HID = 64
FF = 128
OUT_DIM = 80

NC = 2
NS = 16
L = 16

BN = 5000
GRID = N // BN

CHUNK = 128
EPAD_UNIT = NS * CHUNK * 8
E_TOTAL = 800000
EPAD = ((E_TOTAL + EPAD_UNIT - 1) // EPAD_UNIT) * EPAD_UNIT
CROWS = EPAD // CHUNK
CROWS_TILE = CROWS // NS
STAGES = 28
CH_Q = CROWS_TILE // STAGES
GRP = 2
NGRP = CH_Q // GRP
GROWS = GRP * CHUNK
NSLOT = 2

ACC_TILE = 3136
NACC = NS * ACC_TILE
ZB_ROWS = 448
ZB_STEPS = ACC_TILE // ZB_ROWS
WB_ROWS = 224
WB_STEPS = ACC_TILE // WB_ROWS


def _dot(a, b):
    return jnp.dot(a, b, preferred_element_type=jnp.float32)


def _colmax8(v):
    return jnp.broadcast_to(jnp.max(v, axis=0, keepdims=True), (8, HID))


def _k0_body(x_ref, w_ref, b_ref, h_ref, m_ref):
    j = pl.program_id(0)
    h = _dot(x_ref[...], w_ref[...]) + b_ref[0:1, :]
    h_ref[...] = h
    bm = _colmax8(jax.nn.relu(h) + 1e-7)

    @pl.when(j == 0)
    def _():
        m_ref[...] = bm

    @pl.when(j > 0)
    def _():
        m_ref[...] = jnp.maximum(m_ref[...], bm)


def _run_k0(xp, Wlp, bl2):
    return pl.pallas_call(
        _k0_body,
        grid=(GRID,),
        in_specs=[
            pl.BlockSpec((BN, 8), lambda j: (j, 0)),
            pl.BlockSpec((8, HID), lambda j: (0, 0)),
            pl.BlockSpec((8, HID), lambda j: (0, 0)),
        ],
        out_specs=[
            pl.BlockSpec((BN, HID), lambda j: (j, 0)),
            pl.BlockSpec((8, HID), lambda j: (0, 0)),
        ],
        out_shape=[
            jax.ShapeDtypeStruct((N, HID), jnp.float32),
            jax.ShapeDtypeStruct((8, HID), jnp.float32),
        ],
    )(xp, Wlp, bl2)


def _tbl_body(h_ref, m_ref, t_ref):
    v = jax.nn.relu(h_ref[...]) + 1e-7
    w = jnp.exp(v - m_ref[0:1, :])
    wv = w * v
    parts = []
    for cb in range(4):
        parts.append(w[:, cb * L:(cb + 1) * L])
        parts.append(wv[:, cb * L:(cb + 1) * L])
    t_ref[...] = jnp.concatenate(parts, axis=1)


def _run_tbl(h, M):
    return pl.pallas_call(
        _tbl_body,
        grid=(GRID,),
        in_specs=[
            pl.BlockSpec((BN, HID), lambda j: (j, 0)),
            pl.BlockSpec((8, HID), lambda j: (0, 0)),
        ],
        out_specs=[pl.BlockSpec((BN, 2 * HID), lambda j: (j, 0))],
        out_shape=[jax.ShapeDtypeStruct((N, 2 * HID), jnp.float32)],
    )(h, M)[0]


def _sc_body(src_ref, dst_ref, tab_ref, agg_ref,
             accC, sidx, didx, gb, semg, sems, semi):
    c = lax.axis_index("c")
    s = lax.axis_index("s")
    row0 = s * ACC_TILE
    chunk0 = s * CROWS_TILE

    def _fire_gathers(qs, g, slot):
        off = slot * GROWS
        for j in range(GRP):
            r = qs * CH_Q + g * GRP + j
            pltpu.async_copy(tab_ref.at[sidx.at[r]],
                             gb.at[pl.ds(off + j * CHUNK, CHUNK)], semg)

    def _fire_scatters(qs, g, slot):
        off = slot * GROWS
        for j in range(GRP):
            r = qs * CH_Q + g * GRP + j
            pltpu.async_copy(gb.at[pl.ds(off + j * CHUNK, CHUNK)],
                             accC.at[didx.at[r]], sems, add=True)

    def _drain(sem, rows):
        pltpu.make_async_copy(tab_ref.at[pl.ds(0, rows)],
                              gb.at[pl.ds(0, rows)], sem).wait()

    def _fire_idx(q, qs):
        r0 = chunk0 + q * CH_Q
        pltpu.async_copy(src_ref.at[pl.ds(r0, CH_Q)],
                         sidx.at[pl.ds(qs * CH_Q, CH_Q)], semi)
        pltpu.async_copy(dst_ref.at[pl.ds(r0, CH_Q)],
                         didx.at[pl.ds(qs * CH_Q, CH_Q)], semi)

    def _drain_idx():
        pltpu.make_async_copy(src_ref.at[pl.ds(0, CH_Q)],
                              sidx.at[pl.ds(0, CH_Q)], semi).wait()
        pltpu.make_async_copy(dst_ref.at[pl.ds(0, CH_Q)],
                              didx.at[pl.ds(0, CH_Q)], semi).wait()

    def _xform(qs, cb):
        for i in range(CH_Q):
            for j2 in range(CHUNK // L):
                sl = sidx[qs * CH_Q + i, pl.ds(j2 * L, L)]
                sidx[qs * CH_Q + i, pl.ds(j2 * L, L)] = sl * 4 + cb

    def _edges(cb):
        _fire_idx(0, 0)

        def stage(q, _):
            qs = lax.rem(q, 2)

            @pl.when(q < STAGES - 1)
            def _():
                _fire_idx(q + 1, 1 - qs)
            _drain_idx()

            @pl.when(qs == 0)
            def _():
                _xform(0, cb)

            @pl.when(qs == 1)
            def _():
                _xform(1, cb)
            _fire_gathers(qs, 0, 0)

            def grp(g, _):
                slot = lax.rem(g, NSLOT)

                @pl.when(g > 0)
                def _():
                    _drain(sems, GROWS)

                @pl.when(g < NGRP - 1)
                def _():
                    _fire_gathers(qs, g + 1, 1 - slot)
                _drain(semg, GROWS)
                _fire_scatters(qs, g, slot)
                return _
            lax.fori_loop(0, NGRP, grp, None)
            _drain(sems, GROWS)
            return _
        lax.fori_loop(0, STAGES, stage, None)

    def _zero():
        def zf(i, _):
            gb[i, 0:L] = jnp.zeros((L,), jnp.float32)
            gb[i, L:2 * L] = jnp.zeros((L,), jnp.float32)
            return _
        lax.fori_loop(0, ZB_ROWS, zf, None)
        for k in range(ZB_STEPS):
            pltpu.async_copy(gb.at[pl.ds(0, ZB_ROWS)],
                             accC.at[pl.ds(row0 + k * ZB_ROWS, ZB_ROWS)], semg)
        _drain(semg, ACC_TILE)

    def _writeback(cb):
        pltpu.async_copy(accC.at[pl.ds(row0, WB_ROWS)],
                         gb.at[pl.ds(0, WB_ROWS)], semg)

        def wr(k, _):
            off = lax.rem(k, 2) * GROWS

            @pl.when(k > 0)
            def _():
                _drain(sems, WB_ROWS)

            @pl.when(k < WB_STEPS - 1)
            def _():
                pltpu.async_copy(
                    accC.at[pl.ds(row0 + (k + 1) * WB_ROWS, WB_ROWS)],
                    gb.at[pl.ds((lax.rem(k, 2) ^ 1) * GROWS, WB_ROWS)], semg)
            _drain(semg, WB_ROWS)
            pltpu.async_copy(
                gb.at[pl.ds(off, WB_ROWS)],
                agg_ref.at[pl.ds(row0 + k * WB_ROWS, WB_ROWS),
                           pl.ds(cb * 2 * L, 2 * L)], sems)
            return _
        lax.fori_loop(0, WB_STEPS, wr, None)
        _drain(sems, WB_ROWS)

    for phase in range(2):
        _zero()
        plsc.subcore_barrier()

        for cc in range(NC):
            cb = 2 * cc + phase

            @pl.when(c == cc)
            def _(cb=cb):
                _edges(cb)
        plsc.subcore_barrier()

        for cc in range(NC):
            cb = 2 * cc + phase

            @pl.when(c == cc)
            def _(cb=cb):
                _writeback(cb)
        plsc.subcore_barrier()


def _run_sc(src2d, dst2d, tab4):
    f = pl.kernel(
        _sc_body,
        out_type=[jax.ShapeDtypeStruct((NACC, 8 * L), jnp.float32)],
        mesh=plsc.VectorSubcoreMesh(core_axis_name="c", subcore_axis_name="s"),
        compiler_params=pltpu.CompilerParams(use_tc_tiling_on_sc=False),
        scratch_types=[
            pltpu.VMEM_SHARED((NACC, 2 * L), jnp.float32),
            pltpu.VMEM((2 * CH_Q, CHUNK), jnp.int32),
            pltpu.VMEM((2 * CH_Q, CHUNK), jnp.int32),
            pltpu.VMEM((NSLOT * GROWS, 2 * L), jnp.float32),
            pltpu.SemaphoreType.DMA,
            pltpu.SemaphoreType.DMA,
            pltpu.SemaphoreType.DMA,
        ],
    )
    return f(src2d, dst2d, tab4)[0]


def _aggr_from(agg_ref, m_ref):
    eps = jnp.maximum(1e-16 * jnp.exp(-m_ref[0:1, :]), 1e-38)
    a = agg_ref[...]
    return jnp.concatenate(
        [a[:, cb * 2 * L + L:cb * 2 * L + 2 * L]
         / (a[:, cb * 2 * L:cb * 2 * L + L] + eps[:, cb * L:(cb + 1) * L])
         for cb in range(4)], axis=1)


def _layer_body(h_ref, m_ref, w1_ref, b1_ref, w2_ref, b2_ref, agg_ref,
                h_out, m_out, *, first):
    j = pl.program_id(0)
    aggr = _aggr_from(agg_ref, m_ref)
    h = h_ref[...]
    base = h if first else jax.nn.relu(h)
    u = base + aggr
    t = _dot(jax.nn.relu(_dot(u, w1_ref[...]) + b1_ref[0:1, :]),
             w2_ref[...]) + b2_ref[0:1, :]
    hnew = jax.nn.relu(t) if first else h + t
    h_out[...] = hnew
    bm = _colmax8(jax.nn.relu(hnew) + 1e-7)

    @pl.when(j == 0)
    def _():
        m_out[...] = bm

    @pl.when(j > 0)
    def _():
        m_out[...] = jnp.maximum(m_out[...], bm)


def _run_layer(h, M, W1, b1, W2, b2, agg, first):
    return pl.pallas_call(
        functools.partial(_layer_body, first=first),
        grid=(GRID,),
        in_specs=[
            pl.BlockSpec((BN, HID), lambda j: (j, 0)),
            pl.BlockSpec((8, HID), lambda j: (0, 0)),
            pl.BlockSpec((HID, FF), lambda j: (0, 0)),
            pl.BlockSpec((8, FF), lambda j: (0, 0)),
            pl.BlockSpec((FF, HID), lambda j: (0, 0)),
            pl.BlockSpec((8, HID), lambda j: (0, 0)),
            pl.BlockSpec((BN, 8 * L), lambda j: (j, 0)),
        ],
        out_specs=[
            pl.BlockSpec((BN, HID), lambda j: (j, 0)),
            pl.BlockSpec((8, HID), lambda j: (0, 0)),
        ],
        out_shape=[
            jax.ShapeDtypeStruct((N, HID), jnp.float32),
            jax.ShapeDtypeStruct((8, HID), jnp.float32),
        ],
    )(h, M, W1, b1, W2, b2, agg)


def _final_body(h_ref, m_ref, w1_ref, b1_ref, w2_ref, b2_ref,
                wh1_ref, bh1_ref, wh2_ref, bh2_ref, agg_ref,
                out_ref, pool_ref):
    j = pl.program_id(0)
    aggr = _aggr_from(agg_ref, m_ref)
    h = h_ref[...]
    u = jax.nn.relu(h) + aggr
    t = _dot(jax.nn.relu(_dot(u, w1_ref[...]) + b1_ref[0:1, :]),
             w2_ref[...]) + b2_ref[0:1, :]
    hnew = h + t
    bm = jnp.broadcast_to(jnp.max(hnew, axis=0, keepdims=True), (8, HID))

    @pl.when(j == 0)
    def _():
        pool_ref[...] = bm

    @pl.when(j > 0)
    def _():
        pool_ref[...] = jnp.maximum(pool_ref[...], bm)

    @pl.when(j == GRID - 1)
    def _():
        pooled = pool_ref[...]
        z = jax.nn.relu(_dot(pooled, wh1_ref[...]) + bh1_ref[0:1, :])
        out_ref[...] = _dot(z, wh2_ref[...]) + bh2_ref[0:1, :]


def _run_final(h, M, W1, b1, W2, b2, Wh1, bh1, Wh2, bh2, agg):
    return pl.pallas_call(
        _final_body,
        grid=(GRID,),
        in_specs=[
            pl.BlockSpec((BN, HID), lambda j: (j, 0)),
            pl.BlockSpec((8, HID), lambda j: (0, 0)),
            pl.BlockSpec((HID, FF), lambda j: (0, 0)),
            pl.BlockSpec((8, FF), lambda j: (0, 0)),
            pl.BlockSpec((FF, HID), lambda j: (0, 0)),
            pl.BlockSpec((8, HID), lambda j: (0, 0)),
            pl.BlockSpec((HID, HID), lambda j: (0, 0)),
            pl.BlockSpec((8, HID), lambda j: (0, 0)),
            pl.BlockSpec((HID, OUT_DIM), lambda j: (0, 0)),
            pl.BlockSpec((8, OUT_DIM), lambda j: (0, 0)),
            pl.BlockSpec((BN, 8 * L), lambda j: (j, 0)),
        ],
        out_specs=[pl.BlockSpec((8, OUT_DIM), lambda j: (0, 0))],
        out_shape=[jax.ShapeDtypeStruct((8, OUT_DIM), jnp.float32)],
        scratch_shapes=[pltpu.VMEM((8, HID), jnp.float32)],
    )(h, M, W1, b1, W2, b2, Wh1, bh1, Wh2, bh2, agg)


def _b8(b):
    return jnp.broadcast_to(b[None, :], (8, b.shape[0]))


def kernel(x, edge_index, Wl, bl, Win1, bin1, Win2, bin2,
           W0_1, b0_1, W0_2, b0_2, W1_1, b1_1, W1_2, b1_2,
           W2_1, b2_1, W2_2, b2_2, Wh1, bh1, Wh2, bh2):
    xp = jnp.pad(x, ((0, 0), (0, 2)))
    Wlp = jnp.pad(Wl, ((0, 2), (0, 0)))
    src = edge_index[0]
    dst = edge_index[1]
    pad = EPAD - src.shape[0]
    src2d = jnp.concatenate(
        [src, jnp.zeros((pad,), jnp.int32)]).reshape(CROWS, CHUNK)
    dst2d = jnp.concatenate(
        [dst, jnp.full((pad,), N, jnp.int32)]).reshape(CROWS, CHUNK)

    h, M = _run_k0(xp, Wlp, _b8(bl))

    layers = [
        (Win1, bin1, Win2, bin2),
        (W0_1, b0_1, W0_2, b0_2),
        (W1_1, b1_1, W1_2, b1_2),
        (W2_1, b2_1, W2_2, b2_2),
    ]
    for li, (W1, b1, W2, b2) in enumerate(layers):
        tab = _run_tbl(h, M)
        tab4 = tab.reshape(4 * N, 2 * L)
        agg = _run_sc(src2d, dst2d, tab4)
        if li < 3:
            h, M = _run_layer(h, M, W1, _b8(b1), W2, _b8(b2), agg,
                              first=(li == 0))
        else:
            out8 = _run_final(h, M, W1, _b8(b1), W2, _b8(b2),
                              Wh1, _b8(bh1), Wh2, _b8(bh2), agg)[0]
    return out8[0:1, :]


# speedup vs baseline: 1.1355x; 1.0116x over previous
import functools

import jax
import jax.numpy as jnp
from jax import lax
from jax.experimental import pallas as pl
from jax.experimental.pallas import tpu as pltpu
from jax.experimental.pallas import tpu_sc as plsc

N = 50000
HID = 64
FF = 128
OUT_DIM = 80

NC = 2
NS = 16
L = 16

BN = 5000
GRID = N // BN

CHUNK = 128
EPAD_UNIT = NS * CHUNK * 8
E_TOTAL = 800000
EPAD = ((E_TOTAL + EPAD_UNIT - 1) // EPAD_UNIT) * EPAD_UNIT
CROWS = EPAD // CHUNK
CROWS_TILE = CROWS // NS
STAGES = 28
CH_Q = CROWS_TILE // STAGES
GRP = 2
NGRP = CH_Q // GRP
GROWS = GRP * CHUNK
NSLOT = 2

ACC_TILE = 3136
NACC = NS * ACC_TILE
ZB_ROWS = 448
ZB_STEPS = ACC_TILE // ZB_ROWS
WB_ROWS = 224
WB_STEPS = ACC_TILE // WB_ROWS


def _dot(a, b):
    return jnp.dot(a, b, preferred_element_type=jnp.float32)


def _colmax8(v):
    return jnp.broadcast_to(jnp.max(v, axis=0, keepdims=True), (8, HID))


EROWS = E_TOTAL // CHUNK


def _pad_body(e_ref, src_ref, dst_ref):
    src_ref[0:EROWS, :] = e_ref[0, :, :]
    src_ref[EROWS:CROWS, :] = jnp.zeros((CROWS - EROWS, CHUNK), jnp.int32)
    dst_ref[0:EROWS, :] = e_ref[1, :, :]
    dst_ref[EROWS:CROWS, :] = jnp.full((CROWS - EROWS, CHUNK), N, jnp.int32)


def _run_pad(e3):
    return pl.pallas_call(
        _pad_body,
        out_shape=[jax.ShapeDtypeStruct((CROWS, CHUNK), jnp.int32)] * 2,
    )(e3)


def _k0_body(x_ref, w_ref, b_ref, h_ref, m_ref):
    j = pl.program_id(0)
    h = _dot(x_ref[...], w_ref[...]) + b_ref[0:1, :]
    h_ref[...] = h
    bm = _colmax8(jax.nn.relu(h) + 1e-7)

    @pl.when(j == 0)
    def _():
        m_ref[...] = bm

    @pl.when(j > 0)
    def _():
        m_ref[...] = jnp.maximum(m_ref[...], bm)


def _run_k0(xp, Wlp, bl2):
    return pl.pallas_call(
        _k0_body,
        grid=(GRID,),
        in_specs=[
            pl.BlockSpec((BN, 8), lambda j: (j, 0)),
            pl.BlockSpec((8, HID), lambda j: (0, 0)),
            pl.BlockSpec((8, HID), lambda j: (0, 0)),
        ],
        out_specs=[
            pl.BlockSpec((BN, HID), lambda j: (j, 0)),
            pl.BlockSpec((8, HID), lambda j: (0, 0)),
        ],
        out_shape=[
            jax.ShapeDtypeStruct((N, HID), jnp.float32),
            jax.ShapeDtypeStruct((8, HID), jnp.float32),
        ],
    )(xp, Wlp, bl2)


def _tbl_body(h_ref, m_ref, t_ref):
    v = jax.nn.relu(h_ref[...]) + 1e-7
    w = jnp.exp(v - m_ref[0:1, :])
    wv = w * v
    parts = []
    for cb in range(4):
        parts.append(w[:, cb * L:(cb + 1) * L])
        parts.append(wv[:, cb * L:(cb + 1) * L])
    t_ref[...] = jnp.concatenate(parts, axis=1)


def _run_tbl(h, M):
    return pl.pallas_call(
        _tbl_body,
        grid=(GRID,),
        in_specs=[
            pl.BlockSpec((BN, HID), lambda j: (j, 0)),
            pl.BlockSpec((8, HID), lambda j: (0, 0)),
        ],
        out_specs=[pl.BlockSpec((BN, 2 * HID), lambda j: (j, 0))],
        out_shape=[jax.ShapeDtypeStruct((N, 2 * HID), jnp.float32)],
    )(h, M)[0]


def _sc_body(src_ref, dst_ref, tab_ref, agg_ref,
             accC, sidx, didx, gb, semg, sems, semi):
    c = lax.axis_index("c")
    s = lax.axis_index("s")
    row0 = s * ACC_TILE
    chunk0 = s * CROWS_TILE

    def _fire_gathers(qs, g, slot):
        off = slot * GROWS
        for j in range(GRP):
            r = qs * CH_Q + g * GRP + j
            pltpu.async_copy(tab_ref.at[sidx.at[r]],
                             gb.at[pl.ds(off + j * CHUNK, CHUNK)], semg)

    def _fire_scatters(qs, g, slot):
        off = slot * GROWS
        for j in range(GRP):
            r = qs * CH_Q + g * GRP + j
            pltpu.async_copy(gb.at[pl.ds(off + j * CHUNK, CHUNK)],
                             accC.at[didx.at[r]], sems, add=True)

    def _drain(sem, rows):
        pltpu.make_async_copy(tab_ref.at[pl.ds(0, rows)],
                              gb.at[pl.ds(0, rows)], sem).wait()

    def _fire_idx(q, qs):
        r0 = chunk0 + q * CH_Q
        pltpu.async_copy(src_ref.at[pl.ds(r0, CH_Q)],
                         sidx.at[pl.ds(qs * CH_Q, CH_Q)], semi)
        pltpu.async_copy(dst_ref.at[pl.ds(r0, CH_Q)],
                         didx.at[pl.ds(qs * CH_Q, CH_Q)], semi)

    def _drain_idx():
        pltpu.make_async_copy(src_ref.at[pl.ds(0, CH_Q)],
                              sidx.at[pl.ds(0, CH_Q)], semi).wait()
        pltpu.make_async_copy(dst_ref.at[pl.ds(0, CH_Q)],
                              didx.at[pl.ds(0, CH_Q)], semi).wait()

    def _xform(qs, cb):
        for i in range(CH_Q):
            for j2 in range(CHUNK // L):
                sl = sidx[qs * CH_Q + i, pl.ds(j2 * L, L)]
                sidx[qs * CH_Q + i, pl.ds(j2 * L, L)] = sl * 4 + cb

    def _edges(cb):
        _fire_idx(0, 0)

        def stage(q, _):
            qs = lax.rem(q, 2)

            @pl.when(q < STAGES - 1)
            def _():
                _fire_idx(q + 1, 1 - qs)
            _drain_idx()

            @pl.when(qs == 0)
            def _():
                _xform(0, cb)

            @pl.when(qs == 1)
            def _():
                _xform(1, cb)
            _fire_gathers(qs, 0, 0)

            def grp(g, _):
                slot = lax.rem(g, NSLOT)

                @pl.when(g > 0)
                def _():
                    _drain(sems, GROWS)

                @pl.when(g < NGRP - 1)
                def _():
                    _fire_gathers(qs, g + 1, 1 - slot)
                _drain(semg, GROWS)
                _fire_scatters(qs, g, slot)
                return _
            lax.fori_loop(0, NGRP, grp, None)
            _drain(sems, GROWS)
            return _
        lax.fori_loop(0, STAGES, stage, None)

    def _zero():
        def zf(i, _):
            gb[i, 0:L] = jnp.zeros((L,), jnp.float32)
            gb[i, L:2 * L] = jnp.zeros((L,), jnp.float32)
            return _
        lax.fori_loop(0, ZB_ROWS, zf, None)
        for k in range(ZB_STEPS):
            pltpu.async_copy(gb.at[pl.ds(0, ZB_ROWS)],
                             accC.at[pl.ds(row0 + k * ZB_ROWS, ZB_ROWS)], semg)
        _drain(semg, ACC_TILE)

    def _writeback(cb):
        pltpu.async_copy(accC.at[pl.ds(row0, WB_ROWS)],
                         gb.at[pl.ds(0, WB_ROWS)], semg)

        def wr(k, _):
            off = lax.rem(k, 2) * GROWS

            @pl.when(k > 0)
            def _():
                _drain(sems, WB_ROWS)

            @pl.when(k < WB_STEPS - 1)
            def _():
                pltpu.async_copy(
                    accC.at[pl.ds(row0 + (k + 1) * WB_ROWS, WB_ROWS)],
                    gb.at[pl.ds((lax.rem(k, 2) ^ 1) * GROWS, WB_ROWS)], semg)
            _drain(semg, WB_ROWS)
            pltpu.async_copy(
                gb.at[pl.ds(off, WB_ROWS)],
                agg_ref.at[pl.ds(row0 + k * WB_ROWS, WB_ROWS),
                           pl.ds(cb * 2 * L, 2 * L)], sems)
            return _
        lax.fori_loop(0, WB_STEPS, wr, None)
        _drain(sems, WB_ROWS)

    for phase in range(2):
        _zero()
        plsc.subcore_barrier()

        for cc in range(NC):
            cb = 2 * cc + phase

            @pl.when(c == cc)
            def _(cb=cb):
                _edges(cb)
        plsc.subcore_barrier()

        for cc in range(NC):
            cb = 2 * cc + phase

            @pl.when(c == cc)
            def _(cb=cb):
                _writeback(cb)
        plsc.subcore_barrier()


def _run_sc(src2d, dst2d, tab4):
    f = pl.kernel(
        _sc_body,
        out_type=[jax.ShapeDtypeStruct((NACC, 8 * L), jnp.float32)],
        mesh=plsc.VectorSubcoreMesh(core_axis_name="c", subcore_axis_name="s"),
        compiler_params=pltpu.CompilerParams(use_tc_tiling_on_sc=False),
        scratch_types=[
            pltpu.VMEM_SHARED((NACC, 2 * L), jnp.float32),
            pltpu.VMEM((2 * CH_Q, CHUNK), jnp.int32),
            pltpu.VMEM((2 * CH_Q, CHUNK), jnp.int32),
            pltpu.VMEM((NSLOT * GROWS, 2 * L), jnp.float32),
            pltpu.SemaphoreType.DMA,
            pltpu.SemaphoreType.DMA,
            pltpu.SemaphoreType.DMA,
        ],
    )
    return f(src2d, dst2d, tab4)[0]


def _aggr_from(agg_ref, m_ref):
    eps = jnp.maximum(1e-16 * jnp.exp(-m_ref[0:1, :]), 1e-38)
    a = agg_ref[...]
    return jnp.concatenate(
        [a[:, cb * 2 * L + L:cb * 2 * L + 2 * L]
         / (a[:, cb * 2 * L:cb * 2 * L + L] + eps[:, cb * L:(cb + 1) * L])
         for cb in range(4)], axis=1)


def _layer_body(h_ref, m_ref, w1_ref, b1_ref, w2_ref, b2_ref, agg_ref,
                h_out, m_out, *, first):
    j = pl.program_id(0)
    aggr = _aggr_from(agg_ref, m_ref)
    h = h_ref[...]
    base = h if first else jax.nn.relu(h)
    u = base + aggr
    t = _dot(jax.nn.relu(_dot(u, w1_ref[...]) + b1_ref[0:1, :]),
             w2_ref[...]) + b2_ref[0:1, :]
    hnew = jax.nn.relu(t) if first else h + t
    h_out[...] = hnew
    bm = _colmax8(jax.nn.relu(hnew) + 1e-7)

    @pl.when(j == 0)
    def _():
        m_out[...] = bm

    @pl.when(j > 0)
    def _():
        m_out[...] = jnp.maximum(m_out[...], bm)


def _run_layer(h, M, W1, b1, W2, b2, agg, first):
    return pl.pallas_call(
        functools.partial(_layer_body, first=first),
        grid=(GRID,),
        in_specs=[
            pl.BlockSpec((BN, HID), lambda j: (j, 0)),
            pl.BlockSpec((8, HID), lambda j: (0, 0)),
            pl.BlockSpec((HID, FF), lambda j: (0, 0)),
            pl.BlockSpec((8, FF), lambda j: (0, 0)),
            pl.BlockSpec((FF, HID), lambda j: (0, 0)),
            pl.BlockSpec((8, HID), lambda j: (0, 0)),
            pl.BlockSpec((BN, 8 * L), lambda j: (j, 0)),
        ],
        out_specs=[
            pl.BlockSpec((BN, HID), lambda j: (j, 0)),
            pl.BlockSpec((8, HID), lambda j: (0, 0)),
        ],
        out_shape=[
            jax.ShapeDtypeStruct((N, HID), jnp.float32),
            jax.ShapeDtypeStruct((8, HID), jnp.float32),
        ],
    )(h, M, W1, b1, W2, b2, agg)


def _final_body(h_ref, m_ref, w1_ref, b1_ref, w2_ref, b2_ref,
                wh1_ref, bh1_ref, wh2_ref, bh2_ref, agg_ref,
                out_ref, pool_ref):
    j = pl.program_id(0)
    aggr = _aggr_from(agg_ref, m_ref)
    h = h_ref[...]
    u = jax.nn.relu(h) + aggr
    t = _dot(jax.nn.relu(_dot(u, w1_ref[...]) + b1_ref[0:1, :]),
             w2_ref[...]) + b2_ref[0:1, :]
    hnew = h + t
    bm = jnp.broadcast_to(jnp.max(hnew, axis=0, keepdims=True), (8, HID))

    @pl.when(j == 0)
    def _():
        pool_ref[...] = bm

    @pl.when(j > 0)
    def _():
        pool_ref[...] = jnp.maximum(pool_ref[...], bm)

    @pl.when(j == GRID - 1)
    def _():
        pooled = pool_ref[...]
        z = jax.nn.relu(_dot(pooled, wh1_ref[...]) + bh1_ref[0:1, :])
        out_ref[...] = _dot(z, wh2_ref[...]) + bh2_ref[0:1, :]


def _run_final(h, M, W1, b1, W2, b2, Wh1, bh1, Wh2, bh2, agg):
    return pl.pallas_call(
        _final_body,
        grid=(GRID,),
        in_specs=[
            pl.BlockSpec((BN, HID), lambda j: (j, 0)),
            pl.BlockSpec((8, HID), lambda j: (0, 0)),
            pl.BlockSpec((HID, FF), lambda j: (0, 0)),
            pl.BlockSpec((8, FF), lambda j: (0, 0)),
            pl.BlockSpec((FF, HID), lambda j: (0, 0)),
            pl.BlockSpec((8, HID), lambda j: (0, 0)),
            pl.BlockSpec((HID, HID), lambda j: (0, 0)),
            pl.BlockSpec((8, HID), lambda j: (0, 0)),
            pl.BlockSpec((HID, OUT_DIM), lambda j: (0, 0)),
            pl.BlockSpec((8, OUT_DIM), lambda j: (0, 0)),
            pl.BlockSpec((BN, 8 * L), lambda j: (j, 0)),
        ],
        out_specs=[pl.BlockSpec((8, OUT_DIM), lambda j: (0, 0))],
        out_shape=[jax.ShapeDtypeStruct((8, OUT_DIM), jnp.float32)],
        scratch_shapes=[pltpu.VMEM((8, HID), jnp.float32)],
    )(h, M, W1, b1, W2, b2, Wh1, bh1, Wh2, bh2, agg)


def _b8(b):
    return jnp.broadcast_to(b[None, :], (8, b.shape[0]))


def kernel(x, edge_index, Wl, bl, Win1, bin1, Win2, bin2,
           W0_1, b0_1, W0_2, b0_2, W1_1, b1_1, W1_2, b1_2,
           W2_1, b2_1, W2_2, b2_2, Wh1, bh1, Wh2, bh2):
    xp = jnp.pad(x, ((0, 0), (0, 2)))
    Wlp = jnp.pad(Wl, ((0, 2), (0, 0)))
    src2d, dst2d = _run_pad(edge_index.reshape(2, EROWS, CHUNK))

    h, M = _run_k0(xp, Wlp, _b8(bl))

    layers = [
        (Win1, bin1, Win2, bin2),
        (W0_1, b0_1, W0_2, b0_2),
        (W1_1, b1_1, W1_2, b1_2),
        (W2_1, b2_1, W2_2, b2_2),
    ]
    for li, (W1, b1, W2, b2) in enumerate(layers):
        tab = _run_tbl(h, M)
        tab4 = tab.reshape(4 * N, 2 * L)
        agg = _run_sc(src2d, dst2d, tab4)
        if li < 3:
            h, M = _run_layer(h, M, W1, _b8(b1), W2, _b8(b2), agg,
                              first=(li == 0))
        else:
            out8 = _run_final(h, M, W1, _b8(b1), W2, _b8(b2),
                              Wh1, _b8(bh1), Wh2, _b8(bh2), agg)[0]
    return out8[0:1, :]


# speedup vs baseline: 1.1859x; 1.0444x over previous
import functools

import jax
import jax.numpy as jnp
from jax import lax
from jax.experimental import pallas as pl
from jax.experimental.pallas import tpu as pltpu
from jax.experimental.pallas import tpu_sc as plsc

N = 50000
HID = 64
FF = 128
OUT_DIM = 80

NC = 2
NS = 16
L = 16

BN = 5000
GRID = N // BN

CHUNK = 128
EPAD_UNIT = NS * CHUNK * 8
E_TOTAL = 800000
EPAD = ((E_TOTAL + EPAD_UNIT - 1) // EPAD_UNIT) * EPAD_UNIT
CROWS = EPAD // CHUNK
CROWS_TILE = CROWS // NS
STAGES = 28
CH_Q = CROWS_TILE // STAGES
GRP = 2
NGRP = CH_Q // GRP
GROWS = GRP * CHUNK
NSLOT = 2

ACC_TILE = 3136
NACC = NS * ACC_TILE
ZB_ROWS = 448
ZB_STEPS = ACC_TILE // ZB_ROWS
WB_ROWS = 224
WB_STEPS = ACC_TILE // WB_ROWS


def _dot(a, b):
    return jnp.dot(a, b, preferred_element_type=jnp.float32)


def _colmax8(v):
    return jnp.broadcast_to(jnp.max(v, axis=0, keepdims=True), (8, HID))


EROWS = E_TOTAL // CHUNK


def _pad_body(e_ref, src_ref, dst_ref):
    src_ref[0:EROWS, :] = e_ref[0, :, :]
    src_ref[EROWS:CROWS, :] = jnp.zeros((CROWS - EROWS, CHUNK), jnp.int32)
    dst_ref[0:EROWS, :] = e_ref[1, :, :]
    dst_ref[EROWS:CROWS, :] = jnp.full((CROWS - EROWS, CHUNK), N, jnp.int32)


def _run_pad(e3):
    return pl.pallas_call(
        _pad_body,
        out_shape=[jax.ShapeDtypeStruct((CROWS, CHUNK), jnp.int32)] * 2,
    )(e3)


M_SHIFT = 32.0
EPS32 = 1e-16 * float(jnp.exp(jnp.float32(-M_SHIFT)))


def _pack_table(hnew):
    v = jax.nn.relu(hnew) + 1e-7
    w = jnp.exp(v - M_SHIFT)
    wv = w * v
    parts = []
    for cb in range(4):
        parts.append(w[:, cb * L:(cb + 1) * L])
        parts.append(wv[:, cb * L:(cb + 1) * L])
    return jnp.concatenate(parts, axis=1)


def _k0_body(x_ref, w_ref, b_ref, h_ref, t_ref):
    h = _dot(x_ref[...], w_ref[...]) + b_ref[0:1, :]
    h_ref[...] = h
    t_ref[...] = _pack_table(h)


def _run_k0(xp, Wlp, bl2):
    return pl.pallas_call(
        _k0_body,
        grid=(GRID,),
        in_specs=[
            pl.BlockSpec((BN, 8), lambda j: (j, 0)),
            pl.BlockSpec((8, HID), lambda j: (0, 0)),
            pl.BlockSpec((8, HID), lambda j: (0, 0)),
        ],
        out_specs=[
            pl.BlockSpec((BN, HID), lambda j: (j, 0)),
            pl.BlockSpec((BN, 2 * HID), lambda j: (j, 0)),
        ],
        out_shape=[
            jax.ShapeDtypeStruct((N, HID), jnp.float32),
            jax.ShapeDtypeStruct((N, 2 * HID), jnp.float32),
        ],
    )(xp, Wlp, bl2)


def _sc_body(src_ref, dst_ref, tab_ref, agg_ref,
             accC, sidx, didx, gb, semg, sems, semi):
    c = lax.axis_index("c")
    s = lax.axis_index("s")
    row0 = s * ACC_TILE
    chunk0 = s * CROWS_TILE

    def _fire_gathers(qs, g, slot):
        off = slot * GROWS
        for j in range(GRP):
            r = qs * CH_Q + g * GRP + j
            pltpu.async_copy(tab_ref.at[sidx.at[r]],
                             gb.at[pl.ds(off + j * CHUNK, CHUNK)], semg)

    def _fire_scatters(qs, g, slot):
        off = slot * GROWS
        for j in range(GRP):
            r = qs * CH_Q + g * GRP + j
            pltpu.async_copy(gb.at[pl.ds(off + j * CHUNK, CHUNK)],
                             accC.at[didx.at[r]], sems, add=True)

    def _drain(sem, rows):
        pltpu.make_async_copy(tab_ref.at[pl.ds(0, rows)],
                              gb.at[pl.ds(0, rows)], sem).wait()

    def _fire_idx(q, qs):
        r0 = chunk0 + q * CH_Q
        pltpu.async_copy(src_ref.at[pl.ds(r0, CH_Q)],
                         sidx.at[pl.ds(qs * CH_Q, CH_Q)], semi)
        pltpu.async_copy(dst_ref.at[pl.ds(r0, CH_Q)],
                         didx.at[pl.ds(qs * CH_Q, CH_Q)], semi)

    def _drain_idx():
        pltpu.make_async_copy(src_ref.at[pl.ds(0, CH_Q)],
                              sidx.at[pl.ds(0, CH_Q)], semi).wait()
        pltpu.make_async_copy(dst_ref.at[pl.ds(0, CH_Q)],
                              didx.at[pl.ds(0, CH_Q)], semi).wait()

    def _xform(qs, cb):
        for i in range(CH_Q):
            for j2 in range(CHUNK // L):
                sl = sidx[qs * CH_Q + i, pl.ds(j2 * L, L)]
                sidx[qs * CH_Q + i, pl.ds(j2 * L, L)] = sl * 4 + cb

    def _edges(cb):
        _fire_idx(0, 0)

        def stage(q, _):
            qs = lax.rem(q, 2)

            @pl.when(q < STAGES - 1)
            def _():
                _fire_idx(q + 1, 1 - qs)
            _drain_idx()

            @pl.when(qs == 0)
            def _():
                _xform(0, cb)

            @pl.when(qs == 1)
            def _():
                _xform(1, cb)
            _fire_gathers(qs, 0, 0)

            def grp(g, _):
                slot = lax.rem(g, NSLOT)

                @pl.when(g > 0)
                def _():
                    _drain(sems, GROWS)

                @pl.when(g < NGRP - 1)
                def _():
                    _fire_gathers(qs, g + 1, 1 - slot)
                _drain(semg, GROWS)
                _fire_scatters(qs, g, slot)
                return _
            lax.fori_loop(0, NGRP, grp, None)
            _drain(sems, GROWS)
            return _
        lax.fori_loop(0, STAGES, stage, None)

    def _zero():
        def zf(i, _):
            gb[i, 0:L] = jnp.zeros((L,), jnp.float32)
            gb[i, L:2 * L] = jnp.zeros((L,), jnp.float32)
            return _
        lax.fori_loop(0, ZB_ROWS, zf, None)
        for k in range(ZB_STEPS):
            pltpu.async_copy(gb.at[pl.ds(0, ZB_ROWS)],
                             accC.at[pl.ds(row0 + k * ZB_ROWS, ZB_ROWS)], semg)
        _drain(semg, ACC_TILE)

    def _writeback(cb):
        pltpu.async_copy(accC.at[pl.ds(row0, WB_ROWS)],
                         gb.at[pl.ds(0, WB_ROWS)], semg)

        def wr(k, _):
            off = lax.rem(k, 2) * GROWS

            @pl.when(k > 0)
            def _():
                _drain(sems, WB_ROWS)

            @pl.when(k < WB_STEPS - 1)
            def _():
                pltpu.async_copy(
                    accC.at[pl.ds(row0 + (k + 1) * WB_ROWS, WB_ROWS)],
                    gb.at[pl.ds((lax.rem(k, 2) ^ 1) * GROWS, WB_ROWS)], semg)
            _drain(semg, WB_ROWS)
            pltpu.async_copy(
                gb.at[pl.ds(off, WB_ROWS)],
                agg_ref.at[pl.ds(row0 + k * WB_ROWS, WB_ROWS),
                           pl.ds(cb * 2 * L, 2 * L)], sems)
            return _
        lax.fori_loop(0, WB_STEPS, wr, None)
        _drain(sems, WB_ROWS)

    for phase in range(2):
        _zero()
        plsc.subcore_barrier()

        for cc in range(NC):
            cb = 2 * cc + phase

            @pl.when(c == cc)
            def _(cb=cb):
                _edges(cb)
        plsc.subcore_barrier()

        for cc in range(NC):
            cb = 2 * cc + phase

            @pl.when(c == cc)
            def _(cb=cb):
                _writeback(cb)
        plsc.subcore_barrier()


def _run_sc(src2d, dst2d, tab4):
    f = pl.kernel(
        _sc_body,
        out_type=[jax.ShapeDtypeStruct((NACC, 8 * L), jnp.float32)],
        mesh=plsc.VectorSubcoreMesh(core_axis_name="c", subcore_axis_name="s"),
        compiler_params=pltpu.CompilerParams(use_tc_tiling_on_sc=False),
        scratch_types=[
            pltpu.VMEM_SHARED((NACC, 2 * L), jnp.float32),
            pltpu.VMEM((2 * CH_Q, CHUNK), jnp.int32),
            pltpu.VMEM((2 * CH_Q, CHUNK), jnp.int32),
            pltpu.VMEM((NSLOT * GROWS, 2 * L), jnp.float32),
            pltpu.SemaphoreType.DMA,
            pltpu.SemaphoreType.DMA,
            pltpu.SemaphoreType.DMA,
        ],
    )
    return f(src2d, dst2d, tab4)[0]


def _aggr_from(agg_ref):
    a = agg_ref[...]
    return jnp.concatenate(
        [a[:, cb * 2 * L + L:cb * 2 * L + 2 * L]
         / (a[:, cb * 2 * L:cb * 2 * L + L] + EPS32)
         for cb in range(4)], axis=1)


def _layer_body(h_ref, w1_ref, b1_ref, w2_ref, b2_ref, agg_ref,
                h_out, t_out, *, first):
    aggr = _aggr_from(agg_ref)
    h = h_ref[...]
    base = h if first else jax.nn.relu(h)
    u = base + aggr
    t = _dot(jax.nn.relu(_dot(u, w1_ref[...]) + b1_ref[0:1, :]),
             w2_ref[...]) + b2_ref[0:1, :]
    hnew = jax.nn.relu(t) if first else h + t
    h_out[...] = hnew
    t_out[...] = _pack_table(hnew)


def _run_layer(h, W1, b1, W2, b2, agg, first):
    return pl.pallas_call(
        functools.partial(_layer_body, first=first),
        grid=(GRID,),
        in_specs=[
            pl.BlockSpec((BN, HID), lambda j: (j, 0)),
            pl.BlockSpec((HID, FF), lambda j: (0, 0)),
            pl.BlockSpec((8, FF), lambda j: (0, 0)),
            pl.BlockSpec((FF, HID), lambda j: (0, 0)),
            pl.BlockSpec((8, HID), lambda j: (0, 0)),
            pl.BlockSpec((BN, 8 * L), lambda j: (j, 0)),
        ],
        out_specs=[
            pl.BlockSpec((BN, HID), lambda j: (j, 0)),
            pl.BlockSpec((BN, 2 * HID), lambda j: (j, 0)),
        ],
        out_shape=[
            jax.ShapeDtypeStruct((N, HID), jnp.float32),
            jax.ShapeDtypeStruct((N, 2 * HID), jnp.float32),
        ],
    )(h, W1, b1, W2, b2, agg)


def _final_body(h_ref, w1_ref, b1_ref, w2_ref, b2_ref,
                wh1_ref, bh1_ref, wh2_ref, bh2_ref, agg_ref,
                out_ref, pool_ref):
    j = pl.program_id(0)
    aggr = _aggr_from(agg_ref)
    h = h_ref[...]
    u = jax.nn.relu(h) + aggr
    t = _dot(jax.nn.relu(_dot(u, w1_ref[...]) + b1_ref[0:1, :]),
             w2_ref[...]) + b2_ref[0:1, :]
    hnew = h + t
    bm = jnp.broadcast_to(jnp.max(hnew, axis=0, keepdims=True), (8, HID))

    @pl.when(j == 0)
    def _():
        pool_ref[...] = bm

    @pl.when(j > 0)
    def _():
        pool_ref[...] = jnp.maximum(pool_ref[...], bm)

    @pl.when(j == GRID - 1)
    def _():
        pooled = pool_ref[...]
        z = jax.nn.relu(_dot(pooled, wh1_ref[...]) + bh1_ref[0:1, :])
        out_ref[...] = _dot(z, wh2_ref[...]) + bh2_ref[0:1, :]


def _run_final(h, W1, b1, W2, b2, Wh1, bh1, Wh2, bh2, agg):
    return pl.pallas_call(
        _final_body,
        grid=(GRID,),
        in_specs=[
            pl.BlockSpec((BN, HID), lambda j: (j, 0)),
            pl.BlockSpec((HID, FF), lambda j: (0, 0)),
            pl.BlockSpec((8, FF), lambda j: (0, 0)),
            pl.BlockSpec((FF, HID), lambda j: (0, 0)),
            pl.BlockSpec((8, HID), lambda j: (0, 0)),
            pl.BlockSpec((HID, HID), lambda j: (0, 0)),
            pl.BlockSpec((8, HID), lambda j: (0, 0)),
            pl.BlockSpec((HID, OUT_DIM), lambda j: (0, 0)),
            pl.BlockSpec((8, OUT_DIM), lambda j: (0, 0)),
            pl.BlockSpec((BN, 8 * L), lambda j: (j, 0)),
        ],
        out_specs=[pl.BlockSpec((8, OUT_DIM), lambda j: (0, 0))],
        out_shape=[jax.ShapeDtypeStruct((8, OUT_DIM), jnp.float32)],
        scratch_shapes=[pltpu.VMEM((8, HID), jnp.float32)],
    )(h, W1, b1, W2, b2, Wh1, bh1, Wh2, bh2, agg)


def _b8(b):
    return jnp.broadcast_to(b[None, :], (8, b.shape[0]))


def kernel(x, edge_index, Wl, bl, Win1, bin1, Win2, bin2,
           W0_1, b0_1, W0_2, b0_2, W1_1, b1_1, W1_2, b1_2,
           W2_1, b2_1, W2_2, b2_2, Wh1, bh1, Wh2, bh2):
    xp = jnp.pad(x, ((0, 0), (0, 2)))
    Wlp = jnp.pad(Wl, ((0, 2), (0, 0)))
    src2d, dst2d = _run_pad(edge_index.reshape(2, EROWS, CHUNK))

    h, tab = _run_k0(xp, Wlp, _b8(bl))

    layers = [
        (Win1, bin1, Win2, bin2),
        (W0_1, b0_1, W0_2, b0_2),
        (W1_1, b1_1, W1_2, b1_2),
        (W2_1, b2_1, W2_2, b2_2),
    ]
    for li, (W1, b1, W2, b2) in enumerate(layers):
        agg = _run_sc(src2d, dst2d, tab.reshape(4 * N, 2 * L))
        if li < 3:
            h, tab = _run_layer(h, W1, _b8(b1), W2, _b8(b2), agg,
                                first=(li == 0))
        else:
            out8 = _run_final(h, W1, _b8(b1), W2, _b8(b2),
                              Wh1, _b8(bh1), Wh2, _b8(bh2), agg)[0]
    return out8[0:1, :]


# speedup vs baseline: 1.1866x; 1.0006x over previous
import functools

import jax
import jax.numpy as jnp
from jax import lax
from jax.experimental import pallas as pl
from jax.experimental.pallas import tpu as pltpu
from jax.experimental.pallas import tpu_sc as plsc

N = 50000
HID = 64
FF = 128
OUT_DIM = 80

NC = 2
NS = 16
L = 16

BN = 5000
GRID = N // BN

CHUNK = 128
EPAD_UNIT = NS * CHUNK * 8
E_TOTAL = 800000
EPAD = ((E_TOTAL + EPAD_UNIT - 1) // EPAD_UNIT) * EPAD_UNIT
CROWS = EPAD // CHUNK
CROWS_TILE = CROWS // NS
STAGES = 28
CH_Q = CROWS_TILE // STAGES
GRP = 2
NGRP = CH_Q // GRP
GROWS = GRP * CHUNK
NSLOT = 2

ACC_TILE = 3136
NACC = NS * ACC_TILE
ZB_ROWS = 448
ZB_STEPS = ACC_TILE // ZB_ROWS
WB_ROWS = 224
WB_STEPS = ACC_TILE // WB_ROWS


def _dot(a, b):
    return jnp.dot(a, b, preferred_element_type=jnp.float32)


EROWS = E_TOTAL // CHUNK


def _pad_body(e_ref, src_ref, dst_ref):
    src_ref[0:EROWS, :] = e_ref[0, :, :]
    src_ref[EROWS:CROWS, :] = jnp.zeros((CROWS - EROWS, CHUNK), jnp.int32)
    dst_ref[0:EROWS, :] = e_ref[1, :, :]
    dst_ref[EROWS:CROWS, :] = jnp.full((CROWS - EROWS, CHUNK), N, jnp.int32)


def _run_pad(e3):
    return pl.pallas_call(
        _pad_body,
        out_shape=[jax.ShapeDtypeStruct((CROWS, CHUNK), jnp.int32)] * 2,
    )(e3)


M_SHIFT = 32.0
EPS32 = 1e-16 * float(jnp.exp(jnp.float32(-M_SHIFT)))


def _pack_table(hnew):
    v = jax.nn.relu(hnew) + 1e-7
    w = jnp.exp(v - M_SHIFT)
    wv = w * v
    parts = []
    for cb in range(4):
        parts.append(w[:, cb * L:(cb + 1) * L])
        parts.append(wv[:, cb * L:(cb + 1) * L])
    return jnp.concatenate(parts, axis=1)


def _k0_body(x_ref, w_ref, b_ref, h_ref, t_ref):
    h = _dot(x_ref[...], w_ref[...]) + b_ref[0:1, :]
    h_ref[...] = h
    t_ref[...] = _pack_table(h)


def _run_k0(xp, Wlp, bl2):
    return pl.pallas_call(
        _k0_body,
        grid=(GRID,),
        in_specs=[
            pl.BlockSpec((BN, 8), lambda j: (j, 0)),
            pl.BlockSpec((8, HID), lambda j: (0, 0)),
            pl.BlockSpec((8, HID), lambda j: (0, 0)),
        ],
        out_specs=[
            pl.BlockSpec((BN, HID), lambda j: (j, 0)),
            pl.BlockSpec((BN, 2 * HID), lambda j: (j, 0)),
        ],
        out_shape=[
            jax.ShapeDtypeStruct((N, HID), jnp.float32),
            jax.ShapeDtypeStruct((N, 2 * HID), jnp.float32),
        ],
    )(xp, Wlp, bl2)


def _sc_body(src_ref, dst_ref, tab_ref, agg_ref,
             accC, sidx, didx, gb, semg, sems, semi):
    c = lax.axis_index("c")
    s = lax.axis_index("s")
    row0 = s * ACC_TILE
    chunk0 = s * CROWS_TILE

    def _fire_gathers(qs, g, slot):
        off = slot * GROWS
        for j in range(GRP):
            r = qs * CH_Q + g * GRP + j
            pltpu.async_copy(tab_ref.at[sidx.at[r]],
                             gb.at[pl.ds(off + j * CHUNK, CHUNK)], semg)

    def _fire_scatters(qs, g, slot):
        off = slot * GROWS
        for j in range(GRP):
            r = qs * CH_Q + g * GRP + j
            pltpu.async_copy(gb.at[pl.ds(off + j * CHUNK, CHUNK)],
                             accC.at[didx.at[r]], sems, add=True)

    def _drain(sem, rows):
        pltpu.make_async_copy(tab_ref.at[pl.ds(0, rows)],
                              gb.at[pl.ds(0, rows)], sem).wait()

    def _fire_idx(q, qs):
        r0 = chunk0 + q * CH_Q
        pltpu.async_copy(src_ref.at[pl.ds(r0, CH_Q)],
                         sidx.at[pl.ds(qs * CH_Q, CH_Q)], semi)
        pltpu.async_copy(dst_ref.at[pl.ds(r0, CH_Q)],
                         didx.at[pl.ds(qs * CH_Q, CH_Q)], semi)

    def _drain_idx():
        pltpu.make_async_copy(src_ref.at[pl.ds(0, CH_Q)],
                              sidx.at[pl.ds(0, CH_Q)], semi).wait()
        pltpu.make_async_copy(dst_ref.at[pl.ds(0, CH_Q)],
                              didx.at[pl.ds(0, CH_Q)], semi).wait()

    def _xform(qs, cb):
        for i in range(CH_Q):
            for j2 in range(CHUNK // L):
                sl = sidx[qs * CH_Q + i, pl.ds(j2 * L, L)]
                sidx[qs * CH_Q + i, pl.ds(j2 * L, L)] = sl * 4 + cb

    def _edges(cb):
        _fire_idx(0, 0)

        def stage(q, _):
            qs = lax.rem(q, 2)

            @pl.when(q < STAGES - 1)
            def _():
                _fire_idx(q + 1, 1 - qs)
            _drain_idx()

            @pl.when(qs == 0)
            def _():
                _xform(0, cb)

            @pl.when(qs == 1)
            def _():
                _xform(1, cb)
            _fire_gathers(qs, 0, 0)

            def grp(g, _):
                slot = lax.rem(g, NSLOT)

                @pl.when(g > 0)
                def _():
                    _drain(sems, GROWS)

                @pl.when(g < NGRP - 1)
                def _():
                    _fire_gathers(qs, g + 1, 1 - slot)
                _drain(semg, GROWS)
                _fire_scatters(qs, g, slot)
                return _
            lax.fori_loop(0, NGRP, grp, None)
            _drain(sems, GROWS)
            return _
        lax.fori_loop(0, STAGES, stage, None)

    def _zero():
        def zf(i, _):
            gb[i, 0:L] = jnp.zeros((L,), jnp.float32)
            gb[i, L:2 * L] = jnp.zeros((L,), jnp.float32)
            return _
        lax.fori_loop(0, ZB_ROWS, zf, None)
        for k in range(ZB_STEPS):
            pltpu.async_copy(gb.at[pl.ds(0, ZB_ROWS)],
                             accC.at[pl.ds(row0 + k * ZB_ROWS, ZB_ROWS)], semg)
        _drain(semg, ACC_TILE)

    def _writeback(cb):
        pltpu.async_copy(accC.at[pl.ds(row0, WB_ROWS)],
                         gb.at[pl.ds(0, WB_ROWS)], semg)

        def wr(k, _):
            off = lax.rem(k, 2) * GROWS

            @pl.when(k > 0)
            def _():
                _drain(sems, WB_ROWS)

            @pl.when(k < WB_STEPS - 1)
            def _():
                pltpu.async_copy(
                    accC.at[pl.ds(row0 + (k + 1) * WB_ROWS, WB_ROWS)],
                    gb.at[pl.ds((lax.rem(k, 2) ^ 1) * GROWS, WB_ROWS)], semg)
            _drain(semg, WB_ROWS)
            pltpu.async_copy(
                gb.at[pl.ds(off, WB_ROWS)],
                agg_ref.at[pl.ds(row0 + k * WB_ROWS, WB_ROWS),
                           pl.ds(cb * 2 * L, 2 * L)], sems)
            return _
        lax.fori_loop(0, WB_STEPS, wr, None)
        _drain(sems, WB_ROWS)

    for phase in range(2):
        _zero()
        plsc.subcore_barrier()

        for cc in range(NC):
            cb = 2 * cc + phase

            @pl.when(c == cc)
            def _(cb=cb):
                _edges(cb)
        plsc.subcore_barrier()

        for cc in range(NC):
            cb = 2 * cc + phase

            @pl.when(c == cc)
            def _(cb=cb):
                _writeback(cb)
        plsc.subcore_barrier()


def _run_sc(src2d, dst2d, tab4):
    f = pl.kernel(
        _sc_body,
        out_type=[jax.ShapeDtypeStruct((NACC, 8 * L), jnp.float32)],
        mesh=plsc.VectorSubcoreMesh(core_axis_name="c", subcore_axis_name="s"),
        compiler_params=pltpu.CompilerParams(use_tc_tiling_on_sc=False),
        scratch_types=[
            pltpu.VMEM_SHARED((NACC, 2 * L), jnp.float32),
            pltpu.VMEM((2 * CH_Q, CHUNK), jnp.int32),
            pltpu.VMEM((2 * CH_Q, CHUNK), jnp.int32),
            pltpu.VMEM((NSLOT * GROWS, 2 * L), jnp.float32),
            pltpu.SemaphoreType.DMA,
            pltpu.SemaphoreType.DMA,
            pltpu.SemaphoreType.DMA,
        ],
    )
    return f(src2d, dst2d, tab4)[0]


def _aggr_from(agg_ref):
    a = agg_ref[...]
    return jnp.concatenate(
        [a[:, cb * 2 * L + L:cb * 2 * L + 2 * L]
         / (a[:, cb * 2 * L:cb * 2 * L + L] + EPS32)
         for cb in range(4)], axis=1)


def _layer_body(h_ref, w1_ref, b1_ref, w2_ref, b2_ref, agg_ref,
                h_out, t_out, *, first):
    aggr = _aggr_from(agg_ref)
    h = h_ref[...]
    base = h if first else jax.nn.relu(h)
    u = base + aggr
    t = _dot(jax.nn.relu(_dot(u, w1_ref[...]) + b1_ref[0:1, :]),
             w2_ref[...]) + b2_ref[0:1, :]
    hnew = jax.nn.relu(t) if first else h + t
    h_out[...] = hnew
    t_out[...] = _pack_table(hnew)


def _run_layer(h, W1, b1, W2, b2, agg, first):
    return pl.pallas_call(
        functools.partial(_layer_body, first=first),
        grid=(GRID,),
        in_specs=[
            pl.BlockSpec((BN, HID), lambda j: (j, 0)),
            pl.BlockSpec((HID, FF), lambda j: (0, 0)),
            pl.BlockSpec((8, FF), lambda j: (0, 0)),
            pl.BlockSpec((FF, HID), lambda j: (0, 0)),
            pl.BlockSpec((8, HID), lambda j: (0, 0)),
            pl.BlockSpec((BN, 8 * L), lambda j: (j, 0)),
        ],
        out_specs=[
            pl.BlockSpec((BN, HID), lambda j: (j, 0)),
            pl.BlockSpec((BN, 2 * HID), lambda j: (j, 0)),
        ],
        out_shape=[
            jax.ShapeDtypeStruct((N, HID), jnp.float32),
            jax.ShapeDtypeStruct((N, 2 * HID), jnp.float32),
        ],
    )(h, W1, b1, W2, b2, agg)


def _final_body(h_ref, w1_ref, b1_ref, w2_ref, b2_ref,
                wh1_ref, bh1_ref, wh2_ref, bh2_ref, agg_ref,
                out_ref, pool_ref):
    j = pl.program_id(0)
    aggr = _aggr_from(agg_ref)
    h = h_ref[...]
    u = jax.nn.relu(h) + aggr
    t = _dot(jax.nn.relu(_dot(u, w1_ref[...]) + b1_ref[0:1, :]),
             w2_ref[...]) + b2_ref[0:1, :]
    hnew = h + t
    bm = jnp.broadcast_to(jnp.max(hnew, axis=0, keepdims=True), (8, HID))

    @pl.when(j == 0)
    def _():
        pool_ref[...] = bm

    @pl.when(j > 0)
    def _():
        pool_ref[...] = jnp.maximum(pool_ref[...], bm)

    @pl.when(j == GRID - 1)
    def _():
        pooled = pool_ref[...]
        z = jax.nn.relu(_dot(pooled, wh1_ref[...]) + bh1_ref[0:1, :])
        out_ref[...] = _dot(z, wh2_ref[...]) + bh2_ref[0:1, :]


def _run_final(h, W1, b1, W2, b2, Wh1, bh1, Wh2, bh2, agg):
    return pl.pallas_call(
        _final_body,
        grid=(GRID,),
        in_specs=[
            pl.BlockSpec((BN, HID), lambda j: (j, 0)),
            pl.BlockSpec((HID, FF), lambda j: (0, 0)),
            pl.BlockSpec((8, FF), lambda j: (0, 0)),
            pl.BlockSpec((FF, HID), lambda j: (0, 0)),
            pl.BlockSpec((8, HID), lambda j: (0, 0)),
            pl.BlockSpec((HID, HID), lambda j: (0, 0)),
            pl.BlockSpec((8, HID), lambda j: (0, 0)),
            pl.BlockSpec((HID, OUT_DIM), lambda j: (0, 0)),
            pl.BlockSpec((8, OUT_DIM), lambda j: (0, 0)),
            pl.BlockSpec((BN, 8 * L), lambda j: (j, 0)),
        ],
        out_specs=[pl.BlockSpec((8, OUT_DIM), lambda j: (0, 0))],
        out_shape=[jax.ShapeDtypeStruct((8, OUT_DIM), jnp.float32)],
        scratch_shapes=[pltpu.VMEM((8, HID), jnp.float32)],
    )(h, W1, b1, W2, b2, Wh1, bh1, Wh2, bh2, agg)


def _b8(b):
    return jnp.broadcast_to(b[None, :], (8, b.shape[0]))


def kernel(x, edge_index, Wl, bl, Win1, bin1, Win2, bin2,
           W0_1, b0_1, W0_2, b0_2, W1_1, b1_1, W1_2, b1_2,
           W2_1, b2_1, W2_2, b2_2, Wh1, bh1, Wh2, bh2):
    xp = jnp.pad(x, ((0, 0), (0, 2)))
    Wlp = jnp.pad(Wl, ((0, 2), (0, 0)))
    src2d, dst2d = _run_pad(edge_index.reshape(2, EROWS, CHUNK))

    h, tab = _run_k0(xp, Wlp, _b8(bl))

    layers = [
        (Win1, bin1, Win2, bin2),
        (W0_1, b0_1, W0_2, b0_2),
        (W1_1, b1_1, W1_2, b1_2),
        (W2_1, b2_1, W2_2, b2_2),
    ]
    for li, (W1, b1, W2, b2) in enumerate(layers):
        agg = _run_sc(src2d, dst2d, tab.reshape(4 * N, 2 * L))
        if li < 3:
            h, tab = _run_layer(h, W1, _b8(b1), W2, _b8(b2), agg,
                                first=(li == 0))
        else:
            out8 = _run_final(h, W1, _b8(b1), W2, _b8(b2),
                              Wh1, _b8(bh1), Wh2, _b8(bh2), agg)[0]
    return out8[0:1, :]
